# baseline (speedup 1.0000x reference)
import functools

import jax
import jax.numpy as jnp
from jax import lax
from jax.experimental import pallas as pl
from jax.experimental.pallas import tpu as pltpu
from jax.experimental.pallas import tpu_sc as plsc

B = 64
P = 576
D = 768

_SPLIT = 32

_NC = 2
_NS = 16
_NW = _NC * _NS
_BSC = B - _SPLIT
_RPW = (_BSC * P) // _NW
_CPB = 16
_NCH = _RPW // _CPB
_GPR = D // 16
_ROW_OFF = _SPLIT * P


def _tc_body(x_ref, t_ref, o_ref):
    o_ref[...] = x_ref[...] + t_ref[...]


def _sc_body(x_hbm, t_hbm, o_hbm, spos,
             pb0, pb1, xb0, xb1, xb2, xb3,
             ps0, ps1, is0, is1, is2, is3, os0, os1, os2, os3):
    cid = lax.axis_index("c")
    sid = lax.axis_index("s")
    wid = sid * _NC + cid
    row0 = wid * _RPW

    @pl.when(sid == 0)
    def _():
        pltpu.sync_copy(t_hbm, spos)

    plsc.subcore_barrier()

    xbufs = (xb0, xb1, xb2, xb3)
    isems = (is0, is1, is2, is3)
    osems = (os0, os1, os2, os3)
    pbufs = (pb0, pb1)
    psems = (ps0, ps1)

    def xrow(c):
        return _ROW_OFF + row0 + c * _CPB

    def orow(c):
        return row0 + c * _CPB

    def prow(c):
        return (c % (P // _CPB)) * _CPB

    def start_in(c, sx, sp):
        pltpu.make_async_copy(
            x_hbm.at[pl.ds(xrow(c), _CPB), :], xbufs[sx], isems[sx]).start()
        pltpu.make_async_copy(
            spos.at[pl.ds(prow(c), _CPB), :], pbufs[sp], psems[sp]).start()

    start_in(0, 0, 0)
    start_in(1, 1, 1)

    def step(c, sx, sp):
        xbuf, isem, osem = xbufs[sx], isems[sx], osems[sx]
        pbuf, psem = pbufs[sp], psems[sp]

        pltpu.make_async_copy(
            x_hbm.at[pl.ds(xrow(c), _CPB), :], xbuf, isem).wait()
        pltpu.make_async_copy(
            spos.at[pl.ds(prow(c), _CPB), :], pbuf, psem).wait()

        @plsc.parallel_loop(0, _CPB * _GPR, 1, unroll=8)
        def _add_group(i):
            r = i // _GPR
            g = (i % _GPR) * 16
            plsc.addupdate(xbuf.at[r, pl.ds(g, 16)], pbuf[r, pl.ds(g, 16)])

        pltpu.make_async_copy(
            xbuf, o_hbm.at[pl.ds(orow(c), _CPB), :], osem).start()

        sx2 = (sx + 2) % 4

        @pl.when(c + 2 < _NCH)
        def _():
            @pl.when(c >= 2)
            def _():
                pltpu.make_async_copy(
                    xbufs[sx2],
                    o_hbm.at[pl.ds(orow(c - 2), _CPB), :],
                    osems[sx2]).wait()

            start_in(c + 2, sx2, sp)

    def loop(i, carry):
        c0 = i * 4
        step(c0, 0, 0)
        step(c0 + 1, 1, 1)
        step(c0 + 2, 2, 0)
        step(c0 + 3, 3, 1)
        return carry

    lax.fori_loop(0, _NCH // 4, loop, 0)

    for k in range(4):
        c = _NCH - 4 + k
        pltpu.make_async_copy(
            xbufs[c % 4], o_hbm.at[pl.ds(orow(c), _CPB), :],
            osems[c % 4]).wait()


def kernel(encoded_patches, pos_table):
    x2 = encoded_patches.reshape(B * P, D)
    mesh = plsc.VectorSubcoreMesh(core_axis_name="c", subcore_axis_name="s")
    sc_k = functools.partial(
        pl.kernel,
        mesh=mesh,
        out_type=jax.ShapeDtypeStruct((_BSC * P, D), jnp.float32),
        scratch_types=[
            pltpu.VMEM_SHARED((P, D), jnp.float32),
            pltpu.VMEM((_CPB, D), jnp.float32),
            pltpu.VMEM((_CPB, D), jnp.float32),
            pltpu.VMEM((_CPB, D), jnp.float32),
            pltpu.VMEM((_CPB, D), jnp.float32),
            pltpu.VMEM((_CPB, D), jnp.float32),
            pltpu.VMEM((_CPB, D), jnp.float32),
            pltpu.SemaphoreType.DMA,
            pltpu.SemaphoreType.DMA,
            pltpu.SemaphoreType.DMA,
            pltpu.SemaphoreType.DMA,
            pltpu.SemaphoreType.DMA,
            pltpu.SemaphoreType.DMA,
            pltpu.SemaphoreType.DMA,
            pltpu.SemaphoreType.DMA,
            pltpu.SemaphoreType.DMA,
            pltpu.SemaphoreType.DMA,
        ],
    )(_sc_body)
    out_sc = sc_k(x2, pos_table).reshape(_BSC, P, D)

    out_tc = pl.pallas_call(
        _tc_body,
        grid=(_SPLIT,),
        in_specs=[
            pl.BlockSpec((1, P, D), lambda b: (b, 0, 0)),
            pl.BlockSpec((P, D), lambda b: (0, 0)),
        ],
        out_specs=pl.BlockSpec((1, P, D), lambda b: (b, 0, 0)),
        out_shape=jax.ShapeDtypeStruct((_SPLIT, P, D), jnp.float32),
    )(encoded_patches, pos_table)

    return jnp.concatenate([out_tc, out_sc], axis=0)
